# baseline (speedup 1.0000x reference)
import jax
import jax.numpy as jnp
import numpy as np
from jax import lax
from jax.experimental import pallas as pl
from jax.experimental.pallas import tpu as pltpu
from jax.experimental.pallas import tpu_sc as plsc

FH = 14
FW = 14
D_WORD = 300
D_HID = 1024
D_EMB = 2048
SENT_LEN = 45
VOCAB = 10000
B = 32

_NTOK = SENT_LEN * B
_NTOK_PAD = 1536
_ROWS_PER_W = 48
_FR = 224
_NPIX = FH * FW
_NP = B * _FR
_BLK = 896
_NBLK = _NP // _BLK
_NVALID = float(B * _NPIX)
_F32 = jnp.float32


def _sc_gather(table, idx):
    mesh = plsc.VectorSubcoreMesh(core_axis_name="c", subcore_axis_name="s")
    d = table.shape[1]

    def body(table_hbm, idx_hbm, out_hbm, idx_v, rows_v, sem):
        wid = lax.axis_index("s") * 2 + lax.axis_index("c")
        base = wid * _ROWS_PER_W
        pltpu.sync_copy(idx_hbm.at[pl.ds(base, _ROWS_PER_W)], idx_v)
        pltpu.async_copy(table_hbm.at[idx_v], rows_v, sem).wait()
        pltpu.sync_copy(rows_v, out_hbm.at[pl.ds(base, _ROWS_PER_W)])

    f = pl.kernel(
        body,
        mesh=mesh,
        out_type=jax.ShapeDtypeStruct((_NTOK_PAD, d), jnp.float32),
        scratch_types=[
            pltpu.VMEM((_ROWS_PER_W,), jnp.int32),
            pltpu.VMEM((_ROWS_PER_W, d), jnp.float32),
            pltpu.SemaphoreType.DMA,
        ],
    )
    return f(table, idx)


def _proj_body(x_ref, w_ref, b_ref, o_ref):
    x = x_ref[...][:, 0:D_WORD].astype(jnp.bfloat16)
    o_ref[...] = lax.dot_general(
        x, w_ref[...], (((1,), (1,)), ((), ())),
        preferred_element_type=_F32) + b_ref[pl.ds(0, 1), :]


def _proj_call(x, w, b, mblk):
    m, k = x.shape
    n, kw = w.shape
    return pl.pallas_call(
        _proj_body,
        grid=(m // mblk,),
        in_specs=[pl.BlockSpec((mblk, k), lambda i: (i, 0)),
                  pl.BlockSpec((n, kw), lambda i: (0, 0)),
                  pl.BlockSpec((8, n), lambda i: (0, 0))],
        out_specs=pl.BlockSpec((mblk, n), lambda i: (i, 0)),
        out_shape=jax.ShapeDtypeStruct((m, n), jnp.float32),
    )(x, w, b)


def _lstm_body(xw_ref, whh_ref, gb_ref, enc_ref):
    def step(t, hc):
        h, c = hc
        g = xw_ref[pl.ds(pl.multiple_of(t * B, B), B), :] + lax.dot_general(
            h.astype(jnp.bfloat16), whh_ref[...], (((1,), (1,)), ((), ())),
            preferred_element_type=_F32)
        i = jax.nn.sigmoid(g[:, 0:1024])
        f = jax.nn.sigmoid(g[:, 1024:2048])
        gg = jnp.tanh(g[:, 2048:3072])
        o = jax.nn.sigmoid(g[:, 3072:4096])
        c = f * c + i * gg
        return (o * jnp.tanh(c), c)

    h0 = jnp.zeros((B, D_HID), _F32)
    h, _ = lax.fori_loop(0, SENT_LEN, step, (h0, h0))

    gb = gb_ref[...]
    cb = jax.nn.sigmoid(gb[:, 0:1024]) * jnp.tanh(gb[:, 2048:3072])
    hb = jax.nn.sigmoid(gb[:, 3072:4096]) * jnp.tanh(cb)

    e = jnp.concatenate([h, hb], axis=1)
    nrm = jnp.sqrt(jnp.sum(e * e, axis=1, keepdims=True))
    enc_ref[...] = e / jnp.maximum(nrm, 1e-12)


def _lstm_call(x, Wih_f, Whh_f, Wih_b, bf, bb):
    xw = _proj_call(x, Wih_f, bf, 256)
    gb = _proj_call(x[(SENT_LEN - 1) * B:SENT_LEN * B, :], Wih_b, bb, B)
    return pl.pallas_call(
        _lstm_body,
        out_shape=jax.ShapeDtypeStruct((B, D_EMB), jnp.float32),
    )(xw, Whh_f, gb)


def _nproj_body(img_ref, wall_ref, o_ref):
    x = img_ref[0]
    ssq = jnp.sum(x * x, axis=0, keepdims=True)
    xn = (x / jnp.maximum(jnp.sqrt(ssq), 1e-12)).astype(jnp.bfloat16)
    z = lax.dot_general(xn, wall_ref[...], (((0,), (0,)), ((), ())),
                        preferred_element_type=_F32)
    o_ref[...] = jnp.zeros((1, _FR, 9 * 128), _F32)
    o_ref[0, pl.ds(0, _NPIX), :] = z


def _nproj_call(img3, wall):
    return pl.pallas_call(
        _nproj_body,
        grid=(B,),
        in_specs=[pl.BlockSpec((1, 1024, _NPIX), lambda i: (i, 0, 0)),
                  pl.BlockSpec((1024, 9 * 128), lambda i: (0, 0))],
        out_specs=pl.BlockSpec((1, _FR, 9 * 128), lambda i: (i, 0, 0)),
        out_shape=jax.ShapeDtypeStruct((B, _FR, 9 * 128), jnp.float32),
    )(img3, wall)


def _tapsum(zb, ml_ref, mr_ref):
    acc = None
    for t in range(9):
        dj = t % 3
        off = FW * (t // 3 - 1) + (dj - 1)
        r = pltpu.roll(zb[:, t * 128:(t + 1) * 128], (-off) % _BLK, 0)
        if dj == 0:
            r = r * ml_ref[...]
        elif dj == 2:
            r = r * mr_ref[...]
        acc = r if acc is None else acc + r
    return acc


def _accum_sums(i, y, s_ref, s2_ref):
    ps = jnp.broadcast_to(jnp.sum(y, axis=0, keepdims=True), (8, 128))
    ps2 = jnp.broadcast_to(jnp.sum(y * y, axis=0, keepdims=True), (8, 128))

    @pl.when(i == 0)
    def _():
        s_ref[...] = ps
        s2_ref[...] = ps2

    @pl.when(i != 0)
    def _():
        s_ref[...] += ps
        s2_ref[...] += ps2


def _bn_from_sums(x, s_ref, s2_ref, g_ref, b_ref):
    m = s_ref[pl.ds(0, 1), :] / _NVALID
    m2 = s2_ref[pl.ds(0, 1), :] / _NVALID
    var = m2 - m * m
    xn = (x - m) / jnp.sqrt(var + 1e-5)
    return jnp.maximum(xn * g_ref[pl.ds(0, 1), :] + b_ref[pl.ds(0, 1), :],
                       0.0)


def _tap1_body(z_ref, pfr_ref, wcc_ref, mv_ref, ml_ref, mr_ref,
               yraw_ref, s_ref, s2_ref):
    i = pl.program_id(0)
    cm = jnp.dot(pfr_ref[...], wcc_ref[...], preferred_element_type=_F32)
    acc = _tapsum(z_ref[...], ml_ref, mr_ref) + jnp.concatenate(
        [cm, cm, cm, cm], axis=0)
    y = acc * mv_ref[...]
    yraw_ref[...] = y
    _accum_sums(i, y, s_ref, s2_ref)


def _tap1_call(z, pfr, wcc, mv, ml, mr):
    blk = pl.BlockSpec((_BLK, 128), lambda i: (0, 0))
    return pl.pallas_call(
        _tap1_body,
        grid=(_NBLK,),
        in_specs=[pl.BlockSpec((_BLK, 9 * 128), lambda i: (i, 0)),
                  pl.BlockSpec((_FR, 24), lambda i: (0, 0)),
                  pl.BlockSpec((24, 128), lambda i: (0, 0)),
                  blk, blk, blk],
        out_specs=[pl.BlockSpec((_BLK, 128), lambda i: (i, 0)),
                   pl.BlockSpec((8, 128), lambda i: (0, 0)),
                   pl.BlockSpec((8, 128), lambda i: (0, 0))],
        out_shape=[jax.ShapeDtypeStruct((_NP, 128), jnp.float32),
                   jax.ShapeDtypeStruct((8, 128), jnp.float32),
                   jax.ShapeDtypeStruct((8, 128), jnp.float32)],
    )(z, pfr, wcc, mv, ml, mr)


def _mk_res_front(use_res):
    def body(raw_ref, s_ref, s2_ref, g_ref, b_ref, cba_ref, w1c_ref, w1v_ref,
             w2_ref, mv_ref, ml_ref, mr_ref, *rest):
        if use_res:
            res_ref = rest[0]
            v1o_ref, rawo_ref, so_ref, s2o_ref = rest[1:]
        else:
            v1o_ref, rawo_ref, so_ref, s2o_ref = rest
        i = pl.program_id(0)
        mv = mv_ref[...]
        vt = _bn_from_sums(raw_ref[...], s_ref, s2_ref, g_ref, b_ref) * mv
        if use_res:
            vt = vt + res_ref[...]
        cm1 = jnp.dot(cba_ref[...], w1c_ref[...], preferred_element_type=_F32)
        v1 = jnp.maximum(
            jnp.dot(vt, w1v_ref[...], preferred_element_type=_F32)
            + jnp.concatenate([cm1, cm1, cm1, cm1], axis=0), 0.0)
        v1o_ref[...] = v1
        z2 = jnp.dot(v1, w2_ref[...], preferred_element_type=_F32)
        y = _tapsum(z2, ml_ref, mr_ref) * mv
        rawo_ref[...] = y
        _accum_sums(i, y, so_ref, s2o_ref)

    def call(raw, s, s2, g, b, cba, w1c, w1v, w2all, mv, ml, mr, res=None):
        small = pl.BlockSpec((8, 128), lambda i: (0, 0))
        blk = pl.BlockSpec((_BLK, 128), lambda i: (i, 0))
        cblk = pl.BlockSpec((_BLK, 128), lambda i: (0, 0))
        in_specs = [blk, small, small, small, small,
                    pl.BlockSpec((_FR, 8), lambda i: (0, 0)),
                    small,
                    pl.BlockSpec((128, 128), lambda i: (0, 0)),
                    pl.BlockSpec((128, 9 * 128), lambda i: (0, 0)),
                    cblk, cblk, cblk]
        args = [raw, s, s2, g, b, cba, w1c, w1v, w2all, mv, ml, mr]
        if use_res:
            in_specs.append(blk)
            args.append(res)
        return pl.pallas_call(
            body,
            grid=(_NBLK,),
            in_specs=in_specs,
            out_specs=[blk, blk, small, small],
            out_shape=[jax.ShapeDtypeStruct((_NP, 128), jnp.float32),
                       jax.ShapeDtypeStruct((_NP, 128), jnp.float32),
                       jax.ShapeDtypeStruct((8, 128), jnp.float32),
                       jax.ShapeDtypeStruct((8, 128), jnp.float32)],
        )(*args)

    return call


_res_front = _mk_res_front(False)
_res_front_r = _mk_res_front(True)


def _final_body(raw_ref, s_ref, s2_ref, g_ref, b_ref, mv_ref, res_ref,
                out_ref):
    y = _bn_from_sums(raw_ref[...], s_ref, s2_ref, g_ref, b_ref)
    out_ref[...] = y * mv_ref[...] + res_ref[...]


def _final_call(raw, s, s2, g, b, mv, res):
    small = pl.BlockSpec((8, 128), lambda i: (0, 0))
    blk = pl.BlockSpec((_BLK, 128), lambda i: (i, 0))
    return pl.pallas_call(
        _final_body,
        grid=(_NBLK,),
        in_specs=[blk, small, small, small, small,
                  pl.BlockSpec((_BLK, 128), lambda i: (0, 0)), blk],
        out_specs=blk,
        out_shape=jax.ShapeDtypeStruct((_NP, 128), jnp.float32),
    )(raw, s, s2, g, b, mv, res)


def _coord_consts():
    ii = np.arange(_NPIX)
    c0 = (ii / FW - FH // 2) / (FH / 2.0)
    c1 = (ii % FW - FW // 2) / (FW / 2.0)
    coord2d = np.stack([c0, c1], axis=1).reshape(FH, FW, 2).astype(np.float32)
    cbpad = np.pad(coord2d, ((1, 1), (1, 1), (0, 0)))

    pfr = np.zeros((_FR, 24), np.float32)
    cba = np.zeros((_FR, 8), np.float32)
    for i in range(FH):
        for j in range(FW):
            q = FW * i + j
            I, J = i + 1, j + 1
            cba[q, 0] = cbpad[I, J, 0]
            cba[q, 1] = cbpad[I, J, 1]
            cba[q, 2] = 1.0
            pfr[q, 18] = 1.0
            for di in range(3):
                for dj in range(3):
                    for k in range(2):
                        pfr[q, (3 * di + dj) * 2 + k] = (
                            cbpad[I + di - 1, J + dj - 1, k])

    r = np.arange(_BLK)
    q = r % _FR
    mv = (q < _NPIX).astype(np.float32)
    ml = ((q % FW) != 0).astype(np.float32)
    mr = ((q % FW) != FW - 1).astype(np.float32)

    def rep(m):
        return np.ascontiguousarray(
            np.broadcast_to(m[:, None], (_BLK, 128))).astype(np.float32)

    return pfr, cba, rep(mv), rep(ml), rep(mr)


_PFR, _CBA, _MV, _ML, _MR = _coord_consts()


def kernel(que, img, emb, Wih_f, Whh_f, bih_f, bhh_f, Wih_b, Whh_b, bih_b,
           bhh_b, Wc, bc, g0, bt0, W11, b11, W12, b12, g1, bt1, W21, b21,
           W22, b22, g2, bt2):
    f32 = jnp.float32

    idx = jnp.concatenate([
        que.T.astype(jnp.int32).reshape(-1),
        jnp.zeros((_NTOK_PAD - _NTOK,), jnp.int32)])
    table = jnp.pad(emb.astype(f32), ((0, 0), (0, 384 - D_WORD)))
    x = _sc_gather(table, idx)

    def rep8(v):
        return jnp.broadcast_to(v[None, :], (8, v.shape[0]))

    bf16 = jnp.bfloat16
    enc = _lstm_call(x, Wih_f.astype(bf16), Whh_f.astype(bf16),
                     Wih_b.astype(bf16),
                     rep8(bih_f + bhh_f), rep8(bih_b + bhh_b))

    img3 = img.reshape(B, 1024, _NPIX)

    wall = jnp.transpose(Wc[:, :1024], (1, 2, 3, 0)).reshape(
        1024, 9 * 128).astype(bf16)
    wcc = jnp.stack([Wc[:, 1024 + k, di, dj]
                     for di in range(3) for dj in range(3) for k in range(2)],
                    axis=0)
    wcc = jnp.concatenate([wcc, bc[None, :], jnp.zeros((5, 128), f32)], axis=0)

    w11v = W11[:, :128, 0, 0].T
    w11c = jnp.concatenate([W11[:, 128, 0, 0][None], W11[:, 129, 0, 0][None],
                            b11[None], jnp.zeros((5, 128), f32)], axis=0)
    w12 = jnp.transpose(W12, (1, 2, 3, 0)).reshape(128, 9 * 128)
    w21v = W21[:, :128, 0, 0].T
    w21c = jnp.concatenate([W21[:, 128, 0, 0][None], W21[:, 129, 0, 0][None],
                            b21[None], jnp.zeros((5, 128), f32)], axis=0)
    w22 = jnp.transpose(W22, (1, 2, 3, 0)).reshape(128, 9 * 128)

    pfr = jnp.asarray(_PFR)
    cba = jnp.asarray(_CBA)
    mv = jnp.asarray(_MV)
    ml = jnp.asarray(_ML)
    mr = jnp.asarray(_MR)

    z1 = _nproj_call(img3, wall).reshape(_NP, 9 * 128)
    yraw0, s0, s20 = _tap1_call(z1, pfr, wcc, mv, ml, mr)
    v11, raw1, s1, s21 = _res_front(
        yraw0, s0, s20, rep8(g0), rep8(bt0), cba, w11c, w11v, w12,
        mv, ml, mr)
    v12, raw2, s2_, s22 = _res_front_r(
        raw1, s1, s21, rep8(g1), rep8(bt1), cba, w21c, w21v, w22,
        mv, ml, mr, res=v11)
    vout = _final_call(raw2, s2_, s22, rep8(g2), rep8(bt2), mv, v12)

    v = jnp.transpose(
        vout.reshape(B, 16, FW, 128)[:, :FH, :, :], (0, 3, 1, 2))
    return enc, v


# speedup vs baseline: 1.1343x; 1.1343x over previous
import jax
import jax.numpy as jnp
import numpy as np
from jax import lax
from jax.experimental import pallas as pl
from jax.experimental.pallas import tpu as pltpu
from jax.experimental.pallas import tpu_sc as plsc

FH = 14
FW = 14
D_WORD = 300
D_HID = 1024
D_EMB = 2048
SENT_LEN = 45
VOCAB = 10000
B = 32

_NTOK = SENT_LEN * B
_NTOK_PAD = 1536
_ROWS_PER_W = 48
_NPIX = FH * FW
_NP = B * _NPIX
_BLK = 4 * _NPIX
_NBLK = _NP // _BLK
_NVALID = float(_NP)
_F32 = jnp.float32
_BF16 = jnp.bfloat16


def _sc_gather(table, idx):
    mesh = plsc.VectorSubcoreMesh(core_axis_name="c", subcore_axis_name="s")
    d = table.shape[1]

    def body(table_hbm, idx_hbm, out_hbm, idx_v, rows_v, sem):
        wid = lax.axis_index("s") * 2 + lax.axis_index("c")
        base = wid * _ROWS_PER_W
        pltpu.sync_copy(idx_hbm.at[pl.ds(base, _ROWS_PER_W)], idx_v)
        pltpu.async_copy(table_hbm.at[idx_v], rows_v, sem).wait()
        pltpu.sync_copy(rows_v, out_hbm.at[pl.ds(base, _ROWS_PER_W)])

    f = pl.kernel(
        body,
        mesh=mesh,
        out_type=jax.ShapeDtypeStruct((_NTOK_PAD, d), jnp.float32),
        scratch_types=[
            pltpu.VMEM((_ROWS_PER_W,), jnp.int32),
            pltpu.VMEM((_ROWS_PER_W, d), jnp.float32),
            pltpu.SemaphoreType.DMA,
        ],
    )
    return f(table, idx)


_PBLK = 256
_NPROJ = _NTOK_PAD // _PBLK


def _lstm_body(x_ref, wf_ref, bf_ref, whh_ref, wb_ref, bb_ref, enc_ref,
               xw_ref):
    s = pl.program_id(0)

    @pl.when(s < _NPROJ)
    def _():
        st = pl.multiple_of(s * _PBLK, _PBLK)
        xx = x_ref[pl.ds(st, _PBLK), :][:, 0:D_WORD].astype(_BF16)
        xw_ref[pl.ds(st, _PBLK), :] = lax.dot_general(
            xx, wf_ref[...], (((1,), (1,)), ((), ())),
            preferred_element_type=_F32) + bf_ref[pl.ds(0, 1), :]

    @pl.when(s == _NPROJ)
    def _():
        def step(t, hc):
            h, c = hc
            g = xw_ref[pl.ds(pl.multiple_of(t * B, B), B), :] + \
                lax.dot_general(
                    h.astype(_BF16), whh_ref[...], (((1,), (1,)), ((), ())),
                    preferred_element_type=_F32)
            i = jax.nn.sigmoid(g[:, 0:1024])
            f = jax.nn.sigmoid(g[:, 1024:2048])
            gg = jnp.tanh(g[:, 2048:3072])
            o = jax.nn.sigmoid(g[:, 3072:4096])
            c = f * c + i * gg
            return (o * jnp.tanh(c), c)

        h0 = jnp.zeros((B, D_HID), _F32)
        h, _ = lax.fori_loop(0, SENT_LEN, step, (h0, h0))

        xb = x_ref[pl.ds((SENT_LEN - 1) * B, B), :][:, 0:D_WORD].astype(_BF16)
        gb = lax.dot_general(
            xb, wb_ref[...], (((1,), (1,)), ((), ())),
            preferred_element_type=_F32) + bb_ref[pl.ds(0, 1), :]
        cb = jax.nn.sigmoid(gb[:, 0:1024]) * jnp.tanh(gb[:, 2048:3072])
        hb = jax.nn.sigmoid(gb[:, 3072:4096]) * jnp.tanh(cb)

        e = jnp.concatenate([h, hb], axis=1)
        nrm = jnp.sqrt(jnp.sum(e * e, axis=1, keepdims=True))
        enc_ref[...] = e / jnp.maximum(nrm, 1e-12)


def _lstm_call(x, Wih_f, Whh_f, Wih_b, bf, bb):
    full = lambda shape: pl.BlockSpec(shape, lambda s: tuple(0 for _ in shape))
    return pl.pallas_call(
        _lstm_body,
        grid=(_NPROJ + 1,),
        in_specs=[full(x.shape), full(Wih_f.shape), full(bf.shape),
                  full(Whh_f.shape), full(Wih_b.shape), full(bb.shape)],
        out_specs=full((B, D_EMB)),
        out_shape=jax.ShapeDtypeStruct((B, D_EMB), jnp.float32),
        scratch_shapes=[pltpu.VMEM((_NTOK_PAD, 4 * D_HID), jnp.float32)],
    )(x, Wih_f, bf, Whh_f, Wih_b, bb)


def _tapsum(zb, mt_ref, mb_ref, ml_ref, mr_ref):
    acc = None
    for t in range(9):
        di, dj = t // 3, t % 3
        off = FW * (di - 1) + (dj - 1)
        r = pltpu.roll(zb[:, t * 128:(t + 1) * 128], (-off) % _BLK, 0)
        if di == 0:
            r = r * mt_ref[...]
        elif di == 2:
            r = r * mb_ref[...]
        if dj == 0:
            r = r * ml_ref[...]
        elif dj == 2:
            r = r * mr_ref[...]
        acc = r if acc is None else acc + r
    return acc


def _accum_sums(i, y, s_ref, s2_ref):
    ps = jnp.broadcast_to(jnp.sum(y, axis=0, keepdims=True), (8, 128))
    ps2 = jnp.broadcast_to(jnp.sum(y * y, axis=0, keepdims=True), (8, 128))

    @pl.when(i == 0)
    def _():
        s_ref[...] = ps
        s2_ref[...] = ps2

    @pl.when(i != 0)
    def _():
        s_ref[...] += ps
        s2_ref[...] += ps2


def _bn_from_sums(x, s_ref, s2_ref, g_ref, b_ref):
    m = s_ref[pl.ds(0, 1), :] / _NVALID
    m2 = s2_ref[pl.ds(0, 1), :] / _NVALID
    var = m2 - m * m
    xn = (x - m) / jnp.sqrt(var + 1e-5)
    return jnp.maximum(xn * g_ref[pl.ds(0, 1), :] + b_ref[pl.ds(0, 1), :],
                       0.0)


def _conv1_body(img_ref, wall_ref, pfr_ref, wcc_ref, mt_ref, mb_ref, ml_ref,
                mr_ref, yraw_ref, s_ref, s2_ref):
    i = pl.program_id(0)
    x4 = img_ref[...]
    x = jnp.concatenate([x4[0], x4[1], x4[2], x4[3]], axis=1)
    ssq = jnp.sum(x * x, axis=0, keepdims=True)
    xn = (x / jnp.maximum(jnp.sqrt(ssq), 1e-12)).astype(_BF16)
    z = lax.dot_general(xn, wall_ref[...], (((0,), (0,)), ((), ())),
                        preferred_element_type=_F32)
    cm = jnp.dot(pfr_ref[...], wcc_ref[...], preferred_element_type=_F32)
    y = _tapsum(z, mt_ref, mb_ref, ml_ref, mr_ref) + jnp.concatenate(
        [cm, cm, cm, cm], axis=0)
    yraw_ref[...] = y
    _accum_sums(i, y, s_ref, s2_ref)


def _conv1_call(img3, wall, pfr, wcc, mt, mb, ml, mr):
    cblk = pl.BlockSpec((_BLK, 128), lambda i: (0, 0))
    return pl.pallas_call(
        _conv1_body,
        grid=(_NBLK,),
        in_specs=[pl.BlockSpec((4, 1024, _NPIX), lambda i: (i, 0, 0)),
                  pl.BlockSpec((1024, 9 * 128), lambda i: (0, 0)),
                  pl.BlockSpec((_NPIX, 24), lambda i: (0, 0)),
                  pl.BlockSpec((24, 128), lambda i: (0, 0)),
                  cblk, cblk, cblk, cblk],
        out_specs=[pl.BlockSpec((_BLK, 128), lambda i: (i, 0)),
                   pl.BlockSpec((8, 128), lambda i: (0, 0)),
                   pl.BlockSpec((8, 128), lambda i: (0, 0))],
        out_shape=[jax.ShapeDtypeStruct((_NP, 128), jnp.float32),
                   jax.ShapeDtypeStruct((8, 128), jnp.float32),
                   jax.ShapeDtypeStruct((8, 128), jnp.float32)],
    )(img3, wall, pfr, wcc, mt, mb, ml, mr)


def _mk_res_front(use_res):
    def body(raw_ref, s_ref, s2_ref, g_ref, b_ref, cba_ref, w1c_ref, w1v_ref,
             w2_ref, mt_ref, mb_ref, ml_ref, mr_ref, *rest):
        if use_res:
            res_ref = rest[0]
            v1o_ref, rawo_ref, so_ref, s2o_ref = rest[1:]
        else:
            v1o_ref, rawo_ref, so_ref, s2o_ref = rest
        i = pl.program_id(0)
        vt = _bn_from_sums(raw_ref[...], s_ref, s2_ref, g_ref, b_ref)
        if use_res:
            vt = vt + res_ref[...]
        cm1 = jnp.dot(cba_ref[...], w1c_ref[...], preferred_element_type=_F32)
        v1 = jnp.maximum(
            jnp.dot(vt, w1v_ref[...], preferred_element_type=_F32)
            + jnp.concatenate([cm1, cm1, cm1, cm1], axis=0), 0.0)
        v1o_ref[...] = v1
        z2 = jnp.dot(v1, w2_ref[...], preferred_element_type=_F32)
        y = _tapsum(z2, mt_ref, mb_ref, ml_ref, mr_ref)
        rawo_ref[...] = y
        _accum_sums(i, y, so_ref, s2o_ref)

    def call(raw, s, s2, g, b, cba, w1c, w1v, w2all, mt, mb, ml, mr,
             res=None):
        small = pl.BlockSpec((8, 128), lambda i: (0, 0))
        blk = pl.BlockSpec((_BLK, 128), lambda i: (i, 0))
        cblk = pl.BlockSpec((_BLK, 128), lambda i: (0, 0))
        in_specs = [blk, small, small, small, small,
                    pl.BlockSpec((_NPIX, 8), lambda i: (0, 0)),
                    small,
                    pl.BlockSpec((128, 128), lambda i: (0, 0)),
                    pl.BlockSpec((128, 9 * 128), lambda i: (0, 0)),
                    cblk, cblk, cblk, cblk]
        args = [raw, s, s2, g, b, cba, w1c, w1v, w2all, mt, mb, ml, mr]
        if use_res:
            in_specs.append(blk)
            args.append(res)
        return pl.pallas_call(
            body,
            grid=(_NBLK,),
            in_specs=in_specs,
            out_specs=[blk, blk, small, small],
            out_shape=[jax.ShapeDtypeStruct((_NP, 128), jnp.float32),
                       jax.ShapeDtypeStruct((_NP, 128), jnp.float32),
                       jax.ShapeDtypeStruct((8, 128), jnp.float32),
                       jax.ShapeDtypeStruct((8, 128), jnp.float32)],
        )(*args)

    return call


_res_front = _mk_res_front(False)
_res_front_r = _mk_res_front(True)


def _final_body(raw_ref, s_ref, s2_ref, g_ref, b_ref, res_ref, out_ref):
    y = _bn_from_sums(raw_ref[...], s_ref, s2_ref, g_ref, b_ref)
    out_ref[...] = y + res_ref[...]


def _final_call(raw, s, s2, g, b, res):
    small = pl.BlockSpec((8, 128), lambda i: (0, 0))
    blk = pl.BlockSpec((_BLK, 128), lambda i: (i, 0))
    return pl.pallas_call(
        _final_body,
        grid=(_NBLK,),
        in_specs=[blk, small, small, small, small, blk],
        out_specs=blk,
        out_shape=jax.ShapeDtypeStruct((_NP, 128), jnp.float32),
    )(raw, s, s2, g, b, res)


def _coord_consts():
    ii = np.arange(_NPIX)
    c0 = (ii / FW - FH // 2) / (FH / 2.0)
    c1 = (ii % FW - FW // 2) / (FW / 2.0)
    coord2d = np.stack([c0, c1], axis=1).reshape(FH, FW, 2).astype(np.float32)
    cbpad = np.pad(coord2d, ((1, 1), (1, 1), (0, 0)))

    pfr = np.zeros((_NPIX, 24), np.float32)
    cba = np.zeros((_NPIX, 8), np.float32)
    for i in range(FH):
        for j in range(FW):
            q = FW * i + j
            I, J = i + 1, j + 1
            cba[q, 0] = cbpad[I, J, 0]
            cba[q, 1] = cbpad[I, J, 1]
            cba[q, 2] = 1.0
            pfr[q, 18] = 1.0
            for di in range(3):
                for dj in range(3):
                    for k in range(2):
                        pfr[q, (3 * di + dj) * 2 + k] = (
                            cbpad[I + di - 1, J + dj - 1, k])

    r = np.arange(_BLK)
    q = r % _NPIX
    mt = (q >= FW).astype(np.float32)
    mb = (q < _NPIX - FW).astype(np.float32)
    ml = ((q % FW) != 0).astype(np.float32)
    mr = ((q % FW) != FW - 1).astype(np.float32)

    def rep(m):
        return np.ascontiguousarray(
            np.broadcast_to(m[:, None], (_BLK, 128))).astype(np.float32)

    return pfr, cba, rep(mt), rep(mb), rep(ml), rep(mr)


_PFR, _CBA, _MT, _MB, _ML, _MR = _coord_consts()


def kernel(que, img, emb, Wih_f, Whh_f, bih_f, bhh_f, Wih_b, Whh_b, bih_b,
           bhh_b, Wc, bc, g0, bt0, W11, b11, W12, b12, g1, bt1, W21, b21,
           W22, b22, g2, bt2):
    f32 = jnp.float32

    idx = jnp.concatenate([
        que.T.astype(jnp.int32).reshape(-1),
        jnp.zeros((_NTOK_PAD - _NTOK,), jnp.int32)])
    table = jnp.pad(emb.astype(f32), ((0, 0), (0, 384 - D_WORD)))
    x = _sc_gather(table, idx)

    def rep8(v):
        return jnp.broadcast_to(v[None, :], (8, v.shape[0]))

    enc = _lstm_call(x, Wih_f.astype(_BF16), Whh_f.astype(_BF16),
                     Wih_b.astype(_BF16),
                     rep8(bih_f + bhh_f), rep8(bih_b + bhh_b))

    img3 = img.reshape(B, 1024, _NPIX)

    wall = jnp.transpose(Wc[:, :1024], (1, 2, 3, 0)).reshape(
        1024, 9 * 128).astype(_BF16)
    wcc = jnp.stack([Wc[:, 1024 + k, di, dj]
                     for di in range(3) for dj in range(3) for k in range(2)],
                    axis=0)
    wcc = jnp.concatenate([wcc, bc[None, :], jnp.zeros((5, 128), f32)], axis=0)

    w11v = W11[:, :128, 0, 0].T
    w11c = jnp.concatenate([W11[:, 128, 0, 0][None], W11[:, 129, 0, 0][None],
                            b11[None], jnp.zeros((5, 128), f32)], axis=0)
    w12 = jnp.transpose(W12, (1, 2, 3, 0)).reshape(128, 9 * 128)
    w21v = W21[:, :128, 0, 0].T
    w21c = jnp.concatenate([W21[:, 128, 0, 0][None], W21[:, 129, 0, 0][None],
                            b21[None], jnp.zeros((5, 128), f32)], axis=0)
    w22 = jnp.transpose(W22, (1, 2, 3, 0)).reshape(128, 9 * 128)

    pfr = jnp.asarray(_PFR)
    cba = jnp.asarray(_CBA)
    mt = jnp.asarray(_MT)
    mb = jnp.asarray(_MB)
    ml = jnp.asarray(_ML)
    mr = jnp.asarray(_MR)

    yraw0, s0, s20 = _conv1_call(img3, wall, pfr, wcc, mt, mb, ml, mr)
    v11, raw1, s1, s21 = _res_front(
        yraw0, s0, s20, rep8(g0), rep8(bt0), cba, w11c, w11v, w12,
        mt, mb, ml, mr)
    v12, raw2, s2_, s22 = _res_front_r(
        raw1, s1, s21, rep8(g1), rep8(bt1), cba, w21c, w21v, w22,
        mt, mb, ml, mr, res=v11)
    vout = _final_call(raw2, s2_, s22, rep8(g2), rep8(bt2), v12)

    v = jnp.transpose(vout.reshape(B, FH, FW, 128), (0, 3, 1, 2))
    return enc, v


# speedup vs baseline: 1.1671x; 1.0289x over previous
import jax
import jax.numpy as jnp
import numpy as np
from jax import lax
from jax.experimental import pallas as pl
from jax.experimental.pallas import tpu as pltpu
from jax.experimental.pallas import tpu_sc as plsc

FH = 14
FW = 14
D_WORD = 300
D_HID = 1024
D_EMB = 2048
SENT_LEN = 45
VOCAB = 10000
B = 32

_NTOK = SENT_LEN * B
_NTOK_PAD = 1536
_ROWS_PER_W = 48
_NPIX = FH * FW
_NP = B * _NPIX
_BLK = 4 * _NPIX
_NBLK = _NP // _BLK
_NVALID = float(_NP)
_F32 = jnp.float32
_BF16 = jnp.bfloat16


def _sc_gather(table, idx):
    mesh = plsc.VectorSubcoreMesh(core_axis_name="c", subcore_axis_name="s")
    d = table.shape[1]

    def body(table_hbm, idx_hbm, out_hbm, idx_v, rows_v, sem):
        wid = lax.axis_index("s") * 2 + lax.axis_index("c")
        base = wid * _ROWS_PER_W
        pltpu.sync_copy(idx_hbm.at[pl.ds(base, _ROWS_PER_W)], idx_v)
        pltpu.async_copy(table_hbm.at[idx_v], rows_v, sem).wait()
        pltpu.sync_copy(rows_v, out_hbm.at[pl.ds(base, _ROWS_PER_W)])

    f = pl.kernel(
        body,
        mesh=mesh,
        out_type=jax.ShapeDtypeStruct((_NTOK_PAD, d), jnp.float32),
        scratch_types=[
            pltpu.VMEM((_ROWS_PER_W,), jnp.int32),
            pltpu.VMEM((_ROWS_PER_W, d), jnp.float32),
            pltpu.SemaphoreType.DMA,
        ],
    )
    return f(table, idx)


_PBLK = 256
_NPROJ = _NTOK_PAD // _PBLK


def _lstm_body(x_ref, wf_ref, bf_ref, whh_ref, wb_ref, bb_ref, enc_ref,
               xw_ref):
    s = pl.program_id(0)

    @pl.when(s < _NPROJ)
    def _():
        st = pl.multiple_of(s * _PBLK, _PBLK)
        xx = x_ref[pl.ds(st, _PBLK), :][:, 0:D_WORD].astype(_BF16)
        xw_ref[pl.ds(st, _PBLK), :] = lax.dot_general(
            xx, wf_ref[...], (((1,), (1,)), ((), ())),
            preferred_element_type=_F32) + bf_ref[pl.ds(0, 1), :]

    @pl.when(s == _NPROJ)
    def _():
        def step(t, hc):
            h, c = hc
            g = xw_ref[pl.ds(pl.multiple_of(t * B, B), B), :] + \
                lax.dot_general(
                    h.astype(_BF16), whh_ref[...], (((1,), (1,)), ((), ())),
                    preferred_element_type=_F32)
            i = jax.nn.sigmoid(g[:, 0:1024])
            f = jax.nn.sigmoid(g[:, 1024:2048])
            gg = jnp.tanh(g[:, 2048:3072])
            o = jax.nn.sigmoid(g[:, 3072:4096])
            c = f * c + i * gg
            return (o * jnp.tanh(c), c)

        h0 = jnp.zeros((B, D_HID), _F32)
        h, _ = lax.fori_loop(0, SENT_LEN, step, (h0, h0))

        xb = x_ref[pl.ds((SENT_LEN - 1) * B, B), :][:, 0:D_WORD].astype(_BF16)
        gb = lax.dot_general(
            xb, wb_ref[...], (((1,), (1,)), ((), ())),
            preferred_element_type=_F32) + bb_ref[pl.ds(0, 1), :]
        cb = jax.nn.sigmoid(gb[:, 0:1024]) * jnp.tanh(gb[:, 2048:3072])
        hb = jax.nn.sigmoid(gb[:, 3072:4096]) * jnp.tanh(cb)

        e = jnp.concatenate([h, hb], axis=1)
        nrm = jnp.sqrt(jnp.sum(e * e, axis=1, keepdims=True))
        enc_ref[...] = e / jnp.maximum(nrm, 1e-12)


def _lstm_call(x, Wih_f, Whh_f, Wih_b, bf, bb):
    full = lambda shape: pl.BlockSpec(shape, lambda s: tuple(0 for _ in shape))
    return pl.pallas_call(
        _lstm_body,
        grid=(_NPROJ + 1,),
        in_specs=[full(x.shape), full(Wih_f.shape), full(bf.shape),
                  full(Whh_f.shape), full(Wih_b.shape), full(bb.shape)],
        out_specs=full((B, D_EMB)),
        out_shape=jax.ShapeDtypeStruct((B, D_EMB), jnp.float32),
        scratch_shapes=[pltpu.VMEM((_NTOK_PAD, 4 * D_HID), jnp.float32)],
    )(x, Wih_f, bf, Whh_f, Wih_b, bb)


def _tapsum(zb, mt_ref, mb_ref, ml_ref, mr_ref):
    acc = None
    for t in range(9):
        di, dj = t // 3, t % 3
        off = FW * (di - 1) + (dj - 1)
        r = pltpu.roll(zb[:, t * 128:(t + 1) * 128], (-off) % _BLK, 0)
        if di == 0:
            r = r * mt_ref[...]
        elif di == 2:
            r = r * mb_ref[...]
        if dj == 0:
            r = r * ml_ref[...]
        elif dj == 2:
            r = r * mr_ref[...]
        acc = r if acc is None else acc + r
    return acc


def _accum_sums(i, y, s_ref, s2_ref):
    ps = jnp.broadcast_to(jnp.sum(y, axis=0, keepdims=True), (8, 128))
    ps2 = jnp.broadcast_to(jnp.sum(y * y, axis=0, keepdims=True), (8, 128))

    @pl.when(i == 0)
    def _():
        s_ref[...] = ps
        s2_ref[...] = ps2

    @pl.when(i != 0)
    def _():
        s_ref[...] += ps
        s2_ref[...] += ps2


def _bn_from_sums(x, s_ref, s2_ref, g_ref, b_ref):
    m = s_ref[pl.ds(0, 1), :] / _NVALID
    m2 = s2_ref[pl.ds(0, 1), :] / _NVALID
    var = m2 - m * m
    xn = (x - m) / jnp.sqrt(var + 1e-5)
    return jnp.maximum(xn * g_ref[pl.ds(0, 1), :] + b_ref[pl.ds(0, 1), :],
                       0.0)


def _img_body(img_ref, wall_ref, pfr_ref, wcc_ref, cba_ref, w11c_ref,
              w11v_ref, w12_ref, w21c_ref, w21v_ref, w22_ref, g0_ref, b0_ref,
              g1_ref, b1_ref, g2_ref, b2_ref, mt_ref, mb_ref, ml_ref, mr_ref,
              out_ref, a_ref, b_ref2, c_ref, d_ref,
              s0_ref, s20_ref, s1_ref, s21_ref, s2_ref, s22_ref):
    s = pl.program_id(0)
    st = pl.multiple_of((s % _NBLK) * _BLK, _BLK)

    def cm4(pref, wref):
        cm = jnp.dot(pref[...], wref[...], preferred_element_type=_F32)
        return jnp.concatenate([cm, cm, cm, cm], axis=0)

    @pl.when(s < _NBLK)
    def _():
        x4 = img_ref[...]
        x = jnp.concatenate([x4[0], x4[1], x4[2], x4[3]], axis=1)
        ssq = jnp.sum(x * x, axis=0, keepdims=True)
        xn = (x / jnp.maximum(jnp.sqrt(ssq), 1e-12)).astype(_BF16)
        z = lax.dot_general(xn, wall_ref[...], (((0,), (0,)), ((), ())),
                            preferred_element_type=_F32)
        y = _tapsum(z, mt_ref, mb_ref, ml_ref, mr_ref) + cm4(pfr_ref, wcc_ref)
        a_ref[pl.ds(st, _BLK), :] = y
        _accum_sums(s, y, s0_ref, s20_ref)

    def res_phase(raw_ref, sums, gg, bb, w1c, w1v, w2, res_ref, v1o_ref,
                  rawo_ref, so, s2o, first):
        vt = _bn_from_sums(raw_ref[pl.ds(st, _BLK), :], sums[0], sums[1],
                           gg, bb)
        if res_ref is not None:
            vt = vt + res_ref[pl.ds(st, _BLK), :]
        v1 = jnp.maximum(
            jnp.dot(vt, w1v[...], preferred_element_type=_F32)
            + cm4(cba_ref, w1c), 0.0)
        v1o_ref[pl.ds(st, _BLK), :] = v1
        z2 = jnp.dot(v1, w2[...], preferred_element_type=_F32)
        y = _tapsum(z2, mt_ref, mb_ref, ml_ref, mr_ref)
        rawo_ref[pl.ds(st, _BLK), :] = y
        _accum_sums(s - first, y, so, s2o)

    @pl.when((s >= _NBLK) & (s < 2 * _NBLK))
    def _():
        res_phase(a_ref, (s0_ref, s20_ref), g0_ref, b0_ref, w11c_ref,
                  w11v_ref, w12_ref, None, b_ref2, c_ref, s1_ref, s21_ref,
                  _NBLK)

    @pl.when((s >= 2 * _NBLK) & (s < 3 * _NBLK))
    def _():
        res_phase(c_ref, (s1_ref, s21_ref), g1_ref, b1_ref, w21c_ref,
                  w21v_ref, w22_ref, b_ref2, d_ref, a_ref, s2_ref, s22_ref,
                  2 * _NBLK)

    @pl.when(s >= 3 * _NBLK)
    def _():
        out_ref[...] = _bn_from_sums(
            a_ref[pl.ds(st, _BLK), :], s2_ref, s22_ref, g2_ref, b2_ref
        ) + d_ref[pl.ds(st, _BLK), :]


def _img_call(img3, wall, pfr, wcc, cba, w11c, w11v, w12, w21c, w21v, w22,
              g0, b0, g1, b1, g2, b2, mt, mb, ml, mr):
    full = lambda shape: pl.BlockSpec(shape, lambda s: tuple(0 for _ in shape))
    consts = [wall, pfr, wcc, cba, w11c, w11v, w12, w21c, w21v, w22,
              g0, b0, g1, b1, g2, b2, mt, mb, ml, mr]
    return pl.pallas_call(
        _img_body,
        grid=(4 * _NBLK,),
        in_specs=[pl.BlockSpec((4, 1024, _NPIX),
                               lambda s: (jnp.where(s < _NBLK, s, 0), 0, 0))]
        + [full(a.shape) for a in consts],
        out_specs=pl.BlockSpec(
            (_BLK, 128),
            lambda s: (jnp.where(s >= 3 * _NBLK, s % _NBLK, 0), 0)),
        out_shape=jax.ShapeDtypeStruct((_NP, 128), jnp.float32),
        scratch_shapes=[pltpu.VMEM((_NP, 128), jnp.float32)] * 4
        + [pltpu.VMEM((8, 128), jnp.float32)] * 6,
    )(img3, *consts)


def _coord_consts():
    ii = np.arange(_NPIX)
    c0 = (ii / FW - FH // 2) / (FH / 2.0)
    c1 = (ii % FW - FW // 2) / (FW / 2.0)
    coord2d = np.stack([c0, c1], axis=1).reshape(FH, FW, 2).astype(np.float32)
    cbpad = np.pad(coord2d, ((1, 1), (1, 1), (0, 0)))

    pfr = np.zeros((_NPIX, 24), np.float32)
    cba = np.zeros((_NPIX, 8), np.float32)
    for i in range(FH):
        for j in range(FW):
            q = FW * i + j
            I, J = i + 1, j + 1
            cba[q, 0] = cbpad[I, J, 0]
            cba[q, 1] = cbpad[I, J, 1]
            cba[q, 2] = 1.0
            pfr[q, 18] = 1.0
            for di in range(3):
                for dj in range(3):
                    for k in range(2):
                        pfr[q, (3 * di + dj) * 2 + k] = (
                            cbpad[I + di - 1, J + dj - 1, k])

    r = np.arange(_BLK)
    q = r % _NPIX
    mt = (q >= FW).astype(np.float32)
    mb = (q < _NPIX - FW).astype(np.float32)
    ml = ((q % FW) != 0).astype(np.float32)
    mr = ((q % FW) != FW - 1).astype(np.float32)

    def rep(m):
        return np.ascontiguousarray(
            np.broadcast_to(m[:, None], (_BLK, 128))).astype(np.float32)

    return pfr, cba, rep(mt), rep(mb), rep(ml), rep(mr)


_PFR, _CBA, _MT, _MB, _ML, _MR = _coord_consts()


def kernel(que, img, emb, Wih_f, Whh_f, bih_f, bhh_f, Wih_b, Whh_b, bih_b,
           bhh_b, Wc, bc, g0, bt0, W11, b11, W12, b12, g1, bt1, W21, b21,
           W22, b22, g2, bt2):
    f32 = jnp.float32

    idx = jnp.concatenate([
        que.T.astype(jnp.int32).reshape(-1),
        jnp.zeros((_NTOK_PAD - _NTOK,), jnp.int32)])
    table = jnp.pad(emb.astype(f32), ((0, 0), (0, 384 - D_WORD)))
    x = _sc_gather(table, idx)

    def rep8(v):
        return jnp.broadcast_to(v[None, :], (8, v.shape[0]))

    enc = _lstm_call(x, Wih_f.astype(_BF16), Whh_f.astype(_BF16),
                     Wih_b.astype(_BF16),
                     rep8(bih_f + bhh_f), rep8(bih_b + bhh_b))

    img3 = img.reshape(B, 1024, _NPIX)

    wall = jnp.transpose(Wc[:, :1024], (1, 2, 3, 0)).reshape(
        1024, 9 * 128).astype(_BF16)
    wcc = jnp.stack([Wc[:, 1024 + k, di, dj]
                     for di in range(3) for dj in range(3) for k in range(2)],
                    axis=0)
    wcc = jnp.concatenate([wcc, bc[None, :], jnp.zeros((5, 128), f32)], axis=0)

    w11v = W11[:, :128, 0, 0].T
    w11c = jnp.concatenate([W11[:, 128, 0, 0][None], W11[:, 129, 0, 0][None],
                            b11[None], jnp.zeros((5, 128), f32)], axis=0)
    w12 = jnp.transpose(W12, (1, 2, 3, 0)).reshape(128, 9 * 128)
    w21v = W21[:, :128, 0, 0].T
    w21c = jnp.concatenate([W21[:, 128, 0, 0][None], W21[:, 129, 0, 0][None],
                            b21[None], jnp.zeros((5, 128), f32)], axis=0)
    w22 = jnp.transpose(W22, (1, 2, 3, 0)).reshape(128, 9 * 128)

    pfr = jnp.asarray(_PFR)
    cba = jnp.asarray(_CBA)
    mt = jnp.asarray(_MT)
    mb = jnp.asarray(_MB)
    ml = jnp.asarray(_ML)
    mr = jnp.asarray(_MR)

    vout = _img_call(img3, wall, pfr, wcc, cba, w11c, w11v, w12, w21c, w21v,
                     w22, rep8(g0), rep8(bt0), rep8(g1), rep8(bt1),
                     rep8(g2), rep8(bt2), mt, mb, ml, mr)

    v = jnp.transpose(vout.reshape(B, FH, FW, 128), (0, 3, 1, 2))
    return enc, v
